# baseline (speedup 1.0000x reference)
import functools

import jax
import jax.numpy as jnp
from jax.experimental import pallas as pl
from jax.experimental.pallas import tpu as pltpu


def _loss_kernel(q_ref, items_ref, out_ref, norms_ref, *, K, M, CH):
    i = pl.program_id(0)

    @pl.when(i == 0)
    def _compute_norms():
        it = items_ref[...]
        norms_ref[0, :] = jnp.sum(it * it, axis=1)

    qb = q_ref[0]
    T = qb.shape[0]
    qnorm = jnp.sum(qb * qb, axis=1, keepdims=True)

    best_val = None
    for c in range(M // CH):
        ic = items_ref[pl.ds(c * CH, CH), :]
        s = jax.lax.dot_general(
            qb, ic, (((1,), (1,)), ((), ())),
            preferred_element_type=jnp.float32)
        gidx = jax.lax.broadcasted_iota(jnp.int32, (1, CH), 1) + c * CH
        cmax = jnp.max(s, axis=1, keepdims=True)
        mask = s == cmax
        lidx = jnp.min(jnp.where(mask, gidx, M), axis=1, keepdims=True)
        onehot = (gidx == lidx).astype(jnp.float32)
        lnorm = jax.lax.dot_general(
            onehot, norms_ref[0, pl.ds(c * CH, CH)][None, :],
            (((1,), (1,)), ((), ())),
            preferred_element_type=jnp.float32)
        if best_val is None:
            best_val, best_idx, best_norm = cmax, lidx, lnorm
        else:
            better = (cmax > best_val) | ((cmax == best_val) & (lidx < best_idx))
            best_val = jnp.where(better, cmax, best_val)
            best_idx = jnp.where(better, lidx, best_idx)
            best_norm = jnp.where(better, lnorm, best_norm)

    loss = qnorm - 2.0 * best_val + best_norm
    row = jnp.reshape(loss, (T,))

    @pl.when(i % K == 0)
    def _init():
        out_ref[0, 0, :] = row

    @pl.when(i % K != 0)
    def _acc():
        out_ref[0, 0, :] = out_ref[0, 0, :] + row


def kernel(queries, items):
    B, K, T, N = queries.shape
    M = items.shape[0]
    q = queries.reshape(B * K, T, N)
    CH = 4096
    out = pl.pallas_call(
        functools.partial(_loss_kernel, K=K, M=M, CH=CH),
        grid=(B * K,),
        in_specs=[
            pl.BlockSpec((1, T, N), lambda i: (i, 0, 0)),
            pl.BlockSpec((M, N), lambda i: (0, 0)),
        ],
        out_specs=pl.BlockSpec((1, 1, T), lambda i: (i // K, 0, 0)),
        out_shape=jax.ShapeDtypeStruct((B, 1, T), jnp.float32),
        scratch_shapes=[pltpu.VMEM((1, M), jnp.float32)],
    )(q, items)
    return out.reshape(B, T)


# speedup vs baseline: 1.1321x; 1.1321x over previous
import functools

import jax
import jax.numpy as jnp
from jax import lax
from jax.experimental import pallas as pl
from jax.experimental.pallas import tpu as pltpu
from jax.experimental.pallas import tpu_sc as plsc


def _score_kernel(q_ref, items_ref, part_ref, idx_ref, norms_ref, *, K, M, CH):
    i = pl.program_id(0)

    @pl.when(i == 0)
    def _compute_norms():
        it = items_ref[...]
        norms_ref[...] = jnp.sum(it * it, axis=1, keepdims=True)

    qb = q_ref[0]
    T = qb.shape[0]
    qnorm = jnp.sum(qb * qb, axis=1, keepdims=True)

    best_val = None
    for c in range(M // CH):
        ic = items_ref[pl.ds(c * CH, CH), :]
        s = jax.lax.dot_general(
            qb, ic, (((1,), (1,)), ((), ())),
            preferred_element_type=jnp.float32)
        gidx = jax.lax.broadcasted_iota(jnp.int32, (1, CH), 1) + c * CH
        cmax = jnp.max(s, axis=1, keepdims=True)
        mask = s == cmax
        lidx = jnp.min(jnp.where(mask, gidx, M), axis=1, keepdims=True)
        if best_val is None:
            best_val, best_idx = cmax, lidx
        else:
            better = (cmax > best_val) | ((cmax == best_val) & (lidx < best_idx))
            best_val = jnp.where(better, cmax, best_val)
            best_idx = jnp.where(better, lidx, best_idx)

    part = qnorm - 2.0 * best_val
    idx_ref[0] = best_idx

    @pl.when(i % K == 0)
    def _init():
        part_ref[0] = part

    @pl.when(i % K != 0)
    def _acc():
        part_ref[0] = part_ref[0] + part


def _tc_call(queries, items):
    B, K, T, N = queries.shape
    M = items.shape[0]
    q = queries.reshape(B * K, T, N)
    CH = 4096
    part, idx, norms = pl.pallas_call(
        functools.partial(_score_kernel, K=K, M=M, CH=CH),
        grid=(B * K,),
        in_specs=[
            pl.BlockSpec((1, T, N), lambda i: (i, 0, 0)),
            pl.BlockSpec((M, N), lambda i: (0, 0)),
        ],
        out_specs=[
            pl.BlockSpec((1, T, 1), lambda i: (i // K, 0, 0)),
            pl.BlockSpec((1, T, 1), lambda i: (i, 0, 0)),
            pl.BlockSpec((M, 1), lambda i: (0, 0)),
        ],
        out_shape=[
            jax.ShapeDtypeStruct((B, T, 1), jnp.float32),
            jax.ShapeDtypeStruct((B * K, T, 1), jnp.int32),
            jax.ShapeDtypeStruct((M, 1), jnp.float32),
        ],
    )(q, items)
    return part.reshape(B * T), idx.reshape(B * K * T), norms.reshape(M)


def _sc_gather_call(part, idx, norms, B, K, T):
    M = norms.shape[0]
    info = plsc.get_sparse_core_info()
    nw = info.num_cores * info.num_subcores
    span = (B * T) // nw
    wpb = T // span
    assert wpb & (wpb - 1) == 0
    wpb_log2 = wpb.bit_length() - 1
    mesh = plsc.VectorSubcoreMesh(
        core_axis_name="c", subcore_axis_name="s")

    offs = list(range(0, span - 16, 16)) + [span - 16]

    @functools.partial(
        pl.kernel, mesh=mesh,
        compiler_params=pltpu.CompilerParams(needs_layout_passes=False),
        out_type=jax.ShapeDtypeStruct((B * T,), jnp.float32),
        scratch_types=[
            pltpu.VMEM((M,), jnp.float32),
            pltpu.VMEM((span,), jnp.float32),
            pltpu.VMEM((K * span,), jnp.int32),
            pltpu.VMEM((span,), jnp.float32),
        ],
    )
    def sc_body(part_hbm, idx_hbm, norms_hbm, out_hbm, norms_v, part_v,
                idx_v, acc_v):
        wid = lax.axis_index("s") * info.num_cores + lax.axis_index("c")
        j0 = wid * span
        b = lax.shift_right_logical(wid, wpb_log2)
        t0 = jnp.bitwise_and(wid, wpb - 1) * span
        pltpu.sync_copy(norms_hbm, norms_v)
        pltpu.sync_copy(part_hbm.at[pl.ds(j0, span)], part_v)
        for k in range(K):
            r = b * (K * T) + k * T + t0
            pltpu.sync_copy(idx_hbm.at[pl.ds(r, span)],
                            idx_v.at[pl.ds(k * span, span)])
        for o in offs:
            s = part_v[pl.ds(o, 16)]
            for k in range(K):
                g = plsc.load_gather(norms_v, [idx_v[pl.ds(k * span + o, 16)]])
                s = s + g
            acc_v[pl.ds(o, 16)] = s
        pltpu.sync_copy(acc_v, out_hbm.at[pl.ds(j0, span)])

    return sc_body(part, idx, norms)


def kernel(queries, items):
    B, K, T, N = queries.shape
    part, idx, norms = _tc_call(queries, items)
    out = _sc_gather_call(part, idx, norms, B, K, T)
    return out.reshape(B, T)


# speedup vs baseline: 1.2433x; 1.0982x over previous
import functools

import jax
import jax.numpy as jnp
from jax import lax
from jax.experimental import pallas as pl
from jax.experimental.pallas import tpu as pltpu
from jax.experimental.pallas import tpu_sc as plsc


def _score_kernel(q_ref, items_ref, part_ref, idx_ref, norms_ref, *, K, M, CH):
    i = pl.program_id(0)

    @pl.when(i == 0)
    def _compute_norms():
        it = items_ref[...]
        norms_ref[...] = jnp.sum(it * it, axis=1, keepdims=True)

    qb = q_ref[0]
    T = qb.shape[0]
    qnorm = jnp.sum(qb * qb, axis=1, keepdims=True)

    best_val = None
    for c in range(M // CH):
        ic = items_ref[pl.ds(c * CH, CH), :]
        s = jax.lax.dot_general(
            qb, ic, (((1,), (1,)), ((), ())),
            preferred_element_type=jnp.float32)
        gidx = (jax.lax.broadcasted_iota(jnp.int32, (1, CH), 1)
                .astype(jnp.float32) + float(c * CH))
        cmax = jnp.max(s, axis=1, keepdims=True)
        mask = s == cmax
        lidx = jnp.min(jnp.where(mask, gidx, float(M)), axis=1, keepdims=True)
        if best_val is None:
            best_val, best_idx = cmax, lidx
        else:
            better = (cmax > best_val) | ((cmax == best_val) & (lidx < best_idx))
            best_val = jnp.where(better, cmax, best_val)
            best_idx = jnp.where(better, lidx, best_idx)

    part = qnorm - 2.0 * best_val
    idx_ref[0] = best_idx.astype(jnp.int32)

    @pl.when(i % K == 0)
    def _init():
        part_ref[0] = part

    @pl.when(i % K != 0)
    def _acc():
        part_ref[0] = part_ref[0] + part


def _tc_call(queries, items):
    B, K, T, N = queries.shape
    M = items.shape[0]
    q = queries.reshape(B * K, T, N)
    CH = 4096
    part, idx, norms = pl.pallas_call(
        functools.partial(_score_kernel, K=K, M=M, CH=CH),
        grid=(B * K,),
        in_specs=[
            pl.BlockSpec((1, T, N), lambda i: (i, 0, 0)),
            pl.BlockSpec((M, N), lambda i: (0, 0)),
        ],
        out_specs=[
            pl.BlockSpec((1, T, 1), lambda i: (i // K, 0, 0)),
            pl.BlockSpec((1, T, 1), lambda i: (i, 0, 0)),
            pl.BlockSpec((M, 1), lambda i: (0, 0)),
        ],
        out_shape=[
            jax.ShapeDtypeStruct((B, T, 1), jnp.float32),
            jax.ShapeDtypeStruct((B * K, T, 1), jnp.int32),
            jax.ShapeDtypeStruct((M, 1), jnp.float32),
        ],
    )(q, items)
    return part.reshape(B * T), idx.reshape(B * K * T), norms.reshape(M)


def _sc_gather_call(part, idx, norms, B, K, T):
    M = norms.shape[0]
    info = plsc.get_sparse_core_info()
    nw = info.num_cores * info.num_subcores
    span = (B * T) // nw
    wpb = T // span
    assert wpb & (wpb - 1) == 0
    wpb_log2 = wpb.bit_length() - 1
    mesh = plsc.VectorSubcoreMesh(
        core_axis_name="c", subcore_axis_name="s")

    offs = list(range(0, span - 16, 16)) + [span - 16]

    @functools.partial(
        pl.kernel, mesh=mesh,
        compiler_params=pltpu.CompilerParams(needs_layout_passes=False),
        out_type=jax.ShapeDtypeStruct((B * T,), jnp.float32),
        scratch_types=[
            pltpu.VMEM((M,), jnp.float32),
            pltpu.VMEM((span,), jnp.float32),
            pltpu.VMEM((K * span,), jnp.int32),
            pltpu.VMEM((span,), jnp.float32),
        ],
    )
    def sc_body(part_hbm, idx_hbm, norms_hbm, out_hbm, norms_v, part_v,
                idx_v, acc_v):
        wid = lax.axis_index("s") * info.num_cores + lax.axis_index("c")
        j0 = wid * span
        b = lax.shift_right_logical(wid, wpb_log2)
        t0 = jnp.bitwise_and(wid, wpb - 1) * span
        pltpu.sync_copy(norms_hbm, norms_v)
        pltpu.sync_copy(part_hbm.at[pl.ds(j0, span)], part_v)
        for k in range(K):
            r = b * (K * T) + k * T + t0
            pltpu.sync_copy(idx_hbm.at[pl.ds(r, span)],
                            idx_v.at[pl.ds(k * span, span)])
        for o in offs:
            s = part_v[pl.ds(o, 16)]
            for k in range(K):
                g = plsc.load_gather(norms_v, [idx_v[pl.ds(k * span + o, 16)]])
                s = s + g
            acc_v[pl.ds(o, 16)] = s
        pltpu.sync_copy(acc_v, out_hbm.at[pl.ds(j0, span)])

    return sc_body(part, idx, norms)


def kernel(queries, items):
    B, K, T, N = queries.shape
    part, idx, norms = _tc_call(queries, items)
    out = _sc_gather_call(part, idx, norms, B, K, T)
    return out.reshape(B, T)


# speedup vs baseline: 1.2440x; 1.0006x over previous
import functools

import jax
import jax.numpy as jnp
from jax import lax
from jax.experimental import pallas as pl
from jax.experimental.pallas import tpu as pltpu
from jax.experimental.pallas import tpu_sc as plsc


def _score_kernel(q_ref, items_ref, part_ref, idx_ref, norms_ref, *, K, M, CH):
    i = pl.program_id(0)

    @pl.when(i == 0)
    def _compute_norms():
        it = items_ref[...]
        norms_ref[...] = jnp.sum(it * it, axis=1, keepdims=True)

    qb = q_ref[0]
    T = qb.shape[0]
    qnorm = jnp.sum(qb * qb, axis=1, keepdims=True)

    best_val = None
    for c in range(M // CH):
        ic = items_ref[pl.ds(c * CH, CH), :]
        s = jax.lax.dot_general(
            qb, ic, (((1,), (1,)), ((), ())),
            preferred_element_type=jnp.float32)
        gidx = (jax.lax.broadcasted_iota(jnp.int32, (1, CH), 1)
                .astype(jnp.float32) + float(c * CH))
        cmax = jnp.max(s, axis=1, keepdims=True)
        mask = s == cmax
        lidx = jnp.min(jnp.where(mask, gidx, float(M)), axis=1, keepdims=True)
        if best_val is None:
            best_val, best_idx = cmax, lidx
        else:
            better = (cmax > best_val) | ((cmax == best_val) & (lidx < best_idx))
            best_val = jnp.where(better, cmax, best_val)
            best_idx = jnp.where(better, lidx, best_idx)

    part = qnorm - 2.0 * best_val
    idx_ref[0] = best_idx.astype(jnp.int32)

    @pl.when(i % K == 0)
    def _init():
        part_ref[0] = part

    @pl.when(i % K != 0)
    def _acc():
        part_ref[0] = part_ref[0] + part


def _tc_call(queries, items):
    B, K, T, N = queries.shape
    M = items.shape[0]
    q = queries.reshape(B * K, T, N)
    CH = 2048
    part, idx, norms = pl.pallas_call(
        functools.partial(_score_kernel, K=K, M=M, CH=CH),
        grid=(B * K,),
        in_specs=[
            pl.BlockSpec((1, T, N), lambda i: (i, 0, 0)),
            pl.BlockSpec((M, N), lambda i: (0, 0)),
        ],
        out_specs=[
            pl.BlockSpec((1, T, 1), lambda i: (i // K, 0, 0)),
            pl.BlockSpec((1, T, 1), lambda i: (i, 0, 0)),
            pl.BlockSpec((M, 1), lambda i: (0, 0)),
        ],
        out_shape=[
            jax.ShapeDtypeStruct((B, T, 1), jnp.float32),
            jax.ShapeDtypeStruct((B * K, T, 1), jnp.int32),
            jax.ShapeDtypeStruct((M, 1), jnp.float32),
        ],
    )(q, items)
    return part.reshape(B * T), idx.reshape(B * K * T), norms.reshape(M)


def _sc_gather_call(part, idx, norms, B, K, T):
    M = norms.shape[0]
    info = plsc.get_sparse_core_info()
    nw = info.num_cores * info.num_subcores
    span = (B * T) // nw
    wpb = T // span
    assert wpb & (wpb - 1) == 0
    wpb_log2 = wpb.bit_length() - 1
    mesh = plsc.VectorSubcoreMesh(
        core_axis_name="c", subcore_axis_name="s")

    offs = list(range(0, span - 16, 16)) + [span - 16]

    @functools.partial(
        pl.kernel, mesh=mesh,
        compiler_params=pltpu.CompilerParams(needs_layout_passes=False),
        out_type=jax.ShapeDtypeStruct((B * T,), jnp.float32),
        scratch_types=[
            pltpu.VMEM((M,), jnp.float32),
            pltpu.VMEM((span,), jnp.float32),
            pltpu.VMEM((K * span,), jnp.int32),
            pltpu.VMEM((span,), jnp.float32),
        ],
    )
    def sc_body(part_hbm, idx_hbm, norms_hbm, out_hbm, norms_v, part_v,
                idx_v, acc_v):
        wid = lax.axis_index("s") * info.num_cores + lax.axis_index("c")
        j0 = wid * span
        b = lax.shift_right_logical(wid, wpb_log2)
        t0 = jnp.bitwise_and(wid, wpb - 1) * span
        pltpu.sync_copy(norms_hbm, norms_v)
        pltpu.sync_copy(part_hbm.at[pl.ds(j0, span)], part_v)
        for k in range(K):
            r = b * (K * T) + k * T + t0
            pltpu.sync_copy(idx_hbm.at[pl.ds(r, span)],
                            idx_v.at[pl.ds(k * span, span)])
        for o in offs:
            s = part_v[pl.ds(o, 16)]
            for k in range(K):
                g = plsc.load_gather(norms_v, [idx_v[pl.ds(k * span + o, 16)]])
                s = s + g
            acc_v[pl.ds(o, 16)] = s
        pltpu.sync_copy(acc_v, out_hbm.at[pl.ds(j0, span)])

    return sc_body(part, idx, norms)


def kernel(queries, items):
    B, K, T, N = queries.shape
    part, idx, norms = _tc_call(queries, items)
    out = _sc_gather_call(part, idx, norms, B, K, T)
    return out.reshape(B, T)
